# baseline (speedup 1.0000x reference)
import functools

import jax
import jax.numpy as jnp
from jax import lax
from jax.experimental import pallas as pl
from jax.experimental.pallas import tpu as pltpu
from jax.experimental.pallas import tpu_sc as plsc

TOP_K = 30
KPAD = 32
NUM_PE, NUM_RBF = 16, 16
MAX_REL = 32
NPAIR = 25
QCOLS = 32
FEXP = 512
EDGE_TILE = 1024


def _topk_body(x_ref, cat_ref, e_ref, t_ref, d_s, e_s):
    x = x_ref[0]
    n = x[:, 0:3]
    ca = x[:, 3:6]
    cc = x[:, 6:9]
    oo = x[:, 9:12]
    bv = ca - n
    cv = cc - ca
    b0, b1, b2 = bv[:, 0:1], bv[:, 1:2], bv[:, 2:3]
    c0, c1, c2 = cv[:, 0:1], cv[:, 1:2], cv[:, 2:3]
    cr = jnp.concatenate([b1 * c2 - b2 * c1,
                          b2 * c0 - b0 * c2,
                          b0 * c1 - b1 * c0], axis=1)
    cb = -0.58273431 * cr + 0.56802827 * bv - 0.54067466 * cv + ca
    zero_col = jnp.zeros((x.shape[0], 1), jnp.float32)
    t_ref[0] = jnp.concatenate([n, ca, cc, oo, cb, zero_col], axis=1)

    cat = cat_ref[0]
    acc = None
    for c in range(3):
        diff = x[:, 3 + c:4 + c] - cat[c:c + 1, :]
        sq = diff * diff
        acc = sq if acc is None else acc + sq
    d_s[...] = acc

    L = x.shape[0]
    iota_j = lax.broadcasted_iota(jnp.int32, (L, L), 0)

    def body(k, carry):
        dc = d_s[...]
        m = jnp.min(dc, axis=0, keepdims=True)
        cand = jnp.where(dc == m, iota_j, L * 2)
        idx = jnp.min(cand, axis=0, keepdims=True)
        e_s[pl.ds(k, 1), :] = idx
        d_s[...] = jnp.where(iota_j == idx, jnp.float32(jnp.inf), dc)
        return carry

    lax.fori_loop(0, TOP_K, body, 0)
    zrow = jnp.zeros((1, L), jnp.int32)
    e_s[pl.ds(TOP_K, 1), :] = zrow
    e_s[pl.ds(TOP_K + 1, 1), :] = zrow
    e_ref[0] = jnp.transpose(e_s[...], (1, 0))


def _run_topk(Xr, Cat):
    B, L = Xr.shape[0], Xr.shape[1]
    return pl.pallas_call(
        _topk_body,
        grid=(B,),
        in_specs=[
            pl.BlockSpec((1, L, 12), lambda b: (b, 0, 0)),
            pl.BlockSpec((1, 3, L), lambda b: (b, 0, 0)),
        ],
        out_specs=[
            pl.BlockSpec((1, L, KPAD), lambda b: (b, 0, 0)),
            pl.BlockSpec((1, L, 16), lambda b: (b, 0, 0)),
        ],
        out_shape=[
            jax.ShapeDtypeStruct((B, L, KPAD), jnp.int32),
            jax.ShapeDtypeStruct((B, L, 16), jnp.float32),
        ],
        scratch_shapes=[pltpu.VMEM((L, L), jnp.float32),
                        pltpu.VMEM((KPAD, L), jnp.int32)],
    )(Xr, Cat)


def _edge_body(q_ref, s2_ref, w_ref, mus_ref, colv_ref, brow_ref, sc_ref,
               of_ref, o_ref):
    q = q_ref[...]
    dexp = lax.dot_general(q, s2_ref[...], (((1,), (0,)), ((), ())),
                           preferred_element_type=jnp.float32)
    dist = jnp.sqrt(dexp + 1e-6)
    t = dist * 0.8 - mus_ref[...]
    rbf = jnp.exp(-(t * t))
    oneh = (dexp == colv_ref[...]).astype(jnp.float32)
    col = lax.broadcasted_iota(jnp.int32, dexp.shape, 1)
    a = jnp.where(col < NPAIR * NUM_RBF, rbf, oneh)
    acc = lax.dot_general(a.astype(jnp.bfloat16), w_ref[...],
                          (((1,), (0,)), ((), ())),
                          preferred_element_type=jnp.float32) + brow_ref[...]
    m = jnp.mean(acc, axis=1, keepdims=True)
    cen = acc - m
    v = jnp.mean(cen * cen, axis=1, keepdims=True)
    o_ref[...] = cen * lax.rsqrt(v + 1e-5) * sc_ref[...] + of_ref[...]


def _run_edges(Q, S2, W512, mus, colv, brow, scale_row, off_row):
    E = Q.shape[0]
    ntile = E // EDGE_TILE
    return pl.pallas_call(
        _edge_body,
        grid=(ntile,),
        in_specs=[
            pl.BlockSpec((EDGE_TILE, QCOLS), lambda e: (e, 0)),
            pl.BlockSpec((QCOLS, FEXP), lambda e: (0, 0)),
            pl.BlockSpec((FEXP, 128), lambda e: (0, 0)),
            pl.BlockSpec((1, FEXP), lambda e: (0, 0)),
            pl.BlockSpec((1, FEXP), lambda e: (0, 0)),
            pl.BlockSpec((1, 128), lambda e: (0, 0)),
            pl.BlockSpec((1, 128), lambda e: (0, 0)),
            pl.BlockSpec((1, 128), lambda e: (0, 0)),
        ],
        out_specs=pl.BlockSpec((EDGE_TILE, 128), lambda e: (e, 0)),
        out_shape=jax.ShapeDtypeStruct((E, 128), jnp.float32),
    )(Q, S2, W512, mus, colv, brow, scale_row, off_row)


_PAIRS_PY = [(1, 1), (0, 0), (2, 2), (3, 3), (4, 4), (1, 0), (1, 2), (1, 3),
             (1, 4), (0, 2), (0, 3), (0, 4), (4, 2), (4, 3), (3, 2), (0, 1),
             (2, 1), (3, 1), (4, 1), (2, 0), (3, 0), (4, 0), (2, 4), (3, 4),
             (2, 3)]

_NW = 32
_EDGES_PER_W = (4 * 512 * KPAD) // _NW


def _gather_q_sc(T_flat, chain_flat, e_flat):
    nrow = T_flat.shape[0] // 16
    nedge = e_flat.shape[0]
    mesh = plsc.VectorSubcoreMesh(core_axis_name="c", subcore_axis_name="s")

    @functools.partial(
        pl.kernel, mesh=mesh,
        compiler_params=pltpu.CompilerParams(needs_layout_passes=False),
        out_type=jax.ShapeDtypeStruct((nedge * QCOLS,), jnp.float32),
        scratch_types=[
            pltpu.VMEM((nrow * 16,), jnp.float32),
            pltpu.VMEM((nrow,), jnp.int32),
            pltpu.VMEM((_EDGES_PER_W,), jnp.int32),
            pltpu.VMEM((_EDGES_PER_W * QCOLS,), jnp.float32),
        ],
    )
    def k(t_hbm, ch_hbm, e_hbm, q_hbm, tv, chv, ev, qv):
        wid = lax.axis_index("s") * 2 + lax.axis_index("c")
        base = wid * _EDGES_PER_W
        pltpu.sync_copy(t_hbm, tv)
        pltpu.sync_copy(ch_hbm, chv)
        pltpu.sync_copy(e_hbm.at[pl.ds(base, _EDGES_PER_W)], ev)
        b512 = (base >> 14) << 9
        lane = jnp.arange(16, dtype=jnp.int32)
        zz = jnp.zeros((16,), jnp.float32)

        def body(g, carry):
            eg = g * 16 + lane
            j = plsc.load_gather(ev, [eg])
            gcen = (base + eg) >> 5
            gj = j + b512
            cc = plsc.load_gather(chv, [gcen])
            cn = plsc.load_gather(chv, [gj])
            off = gcen - gj
            dcl = jnp.clip(off + MAX_REL, 0, 2 * MAX_REL)
            dd = jnp.where(cc == cn, dcl, 2 * MAX_REL + 1).astype(jnp.float32)
            tc16 = gcen * 16
            tj16 = gj * 16
            ct = [plsc.load_gather(tv, [tc16 + c]) for c in range(15)]
            nb = [plsc.load_gather(tv, [tj16 + c]) for c in range(15)]
            qbase = eg * QCOLS
            for p, (ap, bp) in enumerate(_PAIRS_PY):
                acc = None
                for c in range(3):
                    dif = ct[3 * ap + c] - nb[3 * bp + c]
                    sq = dif * dif
                    acc = sq if acc is None else acc + sq
                plsc.store_scatter(qv, [qbase + p], acc)
            plsc.store_scatter(qv, [qbase + NPAIR], dd)
            for c in range(NPAIR + 1, QCOLS):
                plsc.store_scatter(qv, [qbase + c], zz)
            return carry

        lax.fori_loop(0, _EDGES_PER_W // 16, body, 0)
        pltpu.sync_copy(qv, q_hbm.at[pl.ds(base * QCOLS, _EDGES_PER_W * QCOLS)])

    return k(T_flat, chain_flat, e_flat)


def kernel(X, mask, residue_idx, chain_idx, W_pos, b_pos, W_edge, ln_scale, ln_offset):
    B, L = X.shape[0], X.shape[1]
    K = TOP_K
    Xr = X.reshape(B, L, 12)
    Cat = X[:, :, 1, :].transpose(0, 2, 1)
    E_pad, T = _run_topk(Xr, Cat)
    E_idx = E_pad[:, :, :K]

    Q = _gather_q_sc(T.reshape(-1), chain_idx.reshape(-1),
                     E_pad.reshape(-1)).reshape(B * L * KPAD, QCOLS)

    nd = 2 * MAX_REL + 2
    Wcomb = W_pos @ W_edge[:NUM_PE]
    brow = (b_pos @ W_edge[:NUM_PE]).reshape(1, 128)
    W512 = jnp.zeros((FEXP, 128), jnp.float32)
    W512 = W512.at[:NPAIR * NUM_RBF].set(W_edge[NUM_PE:])
    W512 = W512.at[NPAIR * NUM_RBF:NPAIR * NUM_RBF + nd].set(Wcomb)
    D_mu = jnp.linspace(2.0, 22.0, NUM_RBF)
    mus = jnp.zeros((1, FEXP), jnp.float32)
    mus = mus.at[0, :NPAIR * NUM_RBF].set(jnp.tile(D_mu * 0.8, NPAIR))
    colv = jnp.full((1, FEXP), -1.0, jnp.float32)
    colv = colv.at[0, NPAIR * NUM_RBF:NPAIR * NUM_RBF + nd].set(
        jnp.arange(nd, dtype=jnp.float32))
    S2 = jnp.zeros((QCOLS, FEXP), jnp.float32)
    pcol = jnp.arange(NPAIR * NUM_RBF) // NUM_RBF
    S2 = S2.at[pcol, jnp.arange(NPAIR * NUM_RBF)].set(1.0)
    S2 = S2.at[NPAIR, NPAIR * NUM_RBF:NPAIR * NUM_RBF + nd].set(1.0)

    out = _run_edges(Q, S2, W512.astype(jnp.bfloat16), mus, colv, brow,
                     ln_scale.reshape(1, 128), ln_offset.reshape(1, 128))
    E = out.reshape(B, L, KPAD, 128)[:, :, :K]
    return (E, E_idx)


# speedup vs baseline: 1.0975x; 1.0975x over previous
import functools

import jax
import jax.numpy as jnp
from jax import lax
from jax.experimental import pallas as pl
from jax.experimental.pallas import tpu as pltpu
from jax.experimental.pallas import tpu_sc as plsc

TOP_K = 30
KPAD = 32
NUM_PE, NUM_RBF = 16, 16
MAX_REL = 32
NPAIR = 25
QCOLS = 32
FEXP = 512
EDGE_TILE = 1024


def _topk_body(x_ref, cat_ref, e_ref, t_ref, d_s, e_s):
    x = x_ref[0]
    n = x[:, 0:3]
    ca = x[:, 3:6]
    cc = x[:, 6:9]
    oo = x[:, 9:12]
    bv = ca - n
    cv = cc - ca
    b0, b1, b2 = bv[:, 0:1], bv[:, 1:2], bv[:, 2:3]
    c0, c1, c2 = cv[:, 0:1], cv[:, 1:2], cv[:, 2:3]
    cr = jnp.concatenate([b1 * c2 - b2 * c1,
                          b2 * c0 - b0 * c2,
                          b0 * c1 - b1 * c0], axis=1)
    cb = -0.58273431 * cr + 0.56802827 * bv - 0.54067466 * cv + ca
    zero_col = jnp.zeros((x.shape[0], 1), jnp.float32)
    t_ref[0] = jnp.concatenate([n, ca, cc, oo, cb, zero_col], axis=1)

    cat = cat_ref[0]
    acc = None
    for c in range(3):
        diff = x[:, 3 + c:4 + c] - cat[c:c + 1, :]
        sq = diff * diff
        acc = sq if acc is None else acc + sq
    d_s[...] = acc

    L = x.shape[0]
    iota_j = lax.broadcasted_iota(jnp.int32, (L, L), 0)

    def body(k, carry):
        dc = d_s[...]
        m = jnp.min(dc, axis=0, keepdims=True)
        cand = jnp.where(dc == m, iota_j, L * 2)
        idx = jnp.min(cand, axis=0, keepdims=True)
        e_s[pl.ds(k, 1), :] = idx
        d_s[...] = jnp.where(iota_j == idx, jnp.float32(jnp.inf), dc)
        return carry

    lax.fori_loop(0, TOP_K, body, 0)
    zrow = jnp.zeros((1, L), jnp.int32)
    e_s[pl.ds(TOP_K, 1), :] = zrow
    e_s[pl.ds(TOP_K + 1, 1), :] = zrow
    e_ref[0] = jnp.transpose(e_s[...], (1, 0))


def _run_topk(Xr, Cat):
    B, L = Xr.shape[0], Xr.shape[1]
    return pl.pallas_call(
        _topk_body,
        grid=(B,),
        in_specs=[
            pl.BlockSpec((1, L, 12), lambda b: (b, 0, 0)),
            pl.BlockSpec((1, 3, L), lambda b: (b, 0, 0)),
        ],
        out_specs=[
            pl.BlockSpec((1, L, KPAD), lambda b: (b, 0, 0)),
            pl.BlockSpec((1, L, 16), lambda b: (b, 0, 0)),
        ],
        out_shape=[
            jax.ShapeDtypeStruct((B, L, KPAD), jnp.int32),
            jax.ShapeDtypeStruct((B, L, 16), jnp.float32),
        ],
        scratch_shapes=[pltpu.VMEM((L, L), jnp.float32),
                        pltpu.VMEM((KPAD, L), jnp.int32)],
    )(Xr, Cat)


def _edge_body(q_ref, s2_ref, w_ref, mus_ref, colv_ref, brow_ref, sc_ref,
               of_ref, o_ref):
    lrows = q_ref.shape[1]
    q = q_ref[0].reshape(lrows * KPAD, QCOLS)
    c32 = lax.broadcasted_iota(jnp.int32, q.shape, 1)
    qmix = jnp.where(c32 < NPAIR, jnp.sqrt(q + 1e-6), q)
    dexp = lax.dot_general(qmix, s2_ref[...], (((1,), (0,)), ((), ())),
                           preferred_element_type=jnp.float32)
    t = dexp - mus_ref[...]
    rbf = jnp.exp(-(t * t))
    oneh = (dexp == colv_ref[...]).astype(jnp.float32)
    col = lax.broadcasted_iota(jnp.int32, dexp.shape, 1)
    a = jnp.where(col < NPAIR * NUM_RBF, rbf, oneh)
    acc = lax.dot_general(a.astype(jnp.bfloat16), w_ref[...],
                          (((1,), (0,)), ((), ())),
                          preferred_element_type=jnp.float32) + brow_ref[...]
    m = jnp.mean(acc, axis=1, keepdims=True)
    cen = acc - m
    v = jnp.mean(cen * cen, axis=1, keepdims=True)
    out = cen * lax.rsqrt(v + 1e-5) * sc_ref[...] + of_ref[...]
    o_ref[0] = out.reshape(lrows, KPAD, 128)[:, :TOP_K, :]


def _run_edges(Q4, S2, W512, mus, colv, brow, scale_row, off_row):
    B, L = Q4.shape[0], Q4.shape[1]
    lrows = EDGE_TILE // KPAD
    ntile = L // lrows
    return pl.pallas_call(
        _edge_body,
        grid=(B, ntile),
        in_specs=[
            pl.BlockSpec((1, lrows, KPAD, QCOLS), lambda b, e: (b, e, 0, 0)),
            pl.BlockSpec((QCOLS, FEXP), lambda b, e: (0, 0)),
            pl.BlockSpec((FEXP, 128), lambda b, e: (0, 0)),
            pl.BlockSpec((1, FEXP), lambda b, e: (0, 0)),
            pl.BlockSpec((1, FEXP), lambda b, e: (0, 0)),
            pl.BlockSpec((1, 128), lambda b, e: (0, 0)),
            pl.BlockSpec((1, 128), lambda b, e: (0, 0)),
            pl.BlockSpec((1, 128), lambda b, e: (0, 0)),
        ],
        out_specs=pl.BlockSpec((1, lrows, TOP_K, 128), lambda b, e: (b, e, 0, 0)),
        out_shape=jax.ShapeDtypeStruct((B, L, TOP_K, 128), jnp.float32),
    )(Q4, S2, W512, mus, colv, brow, scale_row, off_row)


_PAIRS_PY = [(1, 1), (0, 0), (2, 2), (3, 3), (4, 4), (1, 0), (1, 2), (1, 3),
             (1, 4), (0, 2), (0, 3), (0, 4), (4, 2), (4, 3), (3, 2), (0, 1),
             (2, 1), (3, 1), (4, 1), (2, 0), (3, 0), (4, 0), (2, 4), (3, 4),
             (2, 3)]

_NW = 32
_EDGES_PER_W = (4 * 512 * KPAD) // _NW


def _gather_q_sc(T_flat, chain_flat, e_flat):
    nrow = T_flat.shape[0] // 16
    nedge = e_flat.shape[0]
    mesh = plsc.VectorSubcoreMesh(core_axis_name="c", subcore_axis_name="s")

    @functools.partial(
        pl.kernel, mesh=mesh,
        compiler_params=pltpu.CompilerParams(needs_layout_passes=False),
        out_type=jax.ShapeDtypeStruct((nedge * QCOLS,), jnp.float32),
        scratch_types=[
            pltpu.VMEM((nrow * 16,), jnp.float32),
            pltpu.VMEM((nrow,), jnp.int32),
            pltpu.VMEM((_EDGES_PER_W,), jnp.int32),
            pltpu.VMEM((_EDGES_PER_W * QCOLS,), jnp.float32),
        ],
    )
    def k(t_hbm, ch_hbm, e_hbm, q_hbm, tv, chv, ev, qv):
        wid = lax.axis_index("s") * 2 + lax.axis_index("c")
        base = wid * _EDGES_PER_W
        pltpu.sync_copy(t_hbm, tv)
        pltpu.sync_copy(ch_hbm, chv)
        pltpu.sync_copy(e_hbm.at[pl.ds(base, _EDGES_PER_W)], ev)
        b512 = (base >> 14) << 9
        lane = jnp.arange(16, dtype=jnp.int32)
        zz = jnp.zeros((16,), jnp.float32)

        def body(g, carry):
            eg = g * 16 + lane
            j = plsc.load_gather(ev, [eg])
            gcen = (base + eg) >> 5
            gj = j + b512
            cc = plsc.load_gather(chv, [gcen])
            cn = plsc.load_gather(chv, [gj])
            off = gcen - gj
            dcl = jnp.clip(off + MAX_REL, 0, 2 * MAX_REL)
            dd = jnp.where(cc == cn, dcl, 2 * MAX_REL + 1).astype(jnp.float32)
            tc16 = gcen * 16
            tj16 = gj * 16
            ct = [plsc.load_gather(tv, [tc16 + c]) for c in range(15)]
            nb = [plsc.load_gather(tv, [tj16 + c]) for c in range(15)]
            qbase = eg * QCOLS
            for p, (ap, bp) in enumerate(_PAIRS_PY):
                acc = None
                for c in range(3):
                    dif = ct[3 * ap + c] - nb[3 * bp + c]
                    sq = dif * dif
                    acc = sq if acc is None else acc + sq
                plsc.store_scatter(qv, [qbase + p], acc)
            plsc.store_scatter(qv, [qbase + NPAIR], dd)
            for c in range(NPAIR + 1, QCOLS):
                plsc.store_scatter(qv, [qbase + c], zz)
            return carry

        lax.fori_loop(0, _EDGES_PER_W // 16, body, 0)
        pltpu.sync_copy(qv, q_hbm.at[pl.ds(base * QCOLS, _EDGES_PER_W * QCOLS)])

    return k(T_flat, chain_flat, e_flat)


def kernel(X, mask, residue_idx, chain_idx, W_pos, b_pos, W_edge, ln_scale, ln_offset):
    B, L = X.shape[0], X.shape[1]
    K = TOP_K
    Xr = X.reshape(B, L, 12)
    Cat = X[:, :, 1, :].transpose(0, 2, 1)
    E_pad, T = _run_topk(Xr, Cat)
    E_idx = E_pad[:, :, :K]

    Q = _gather_q_sc(T.reshape(-1), chain_idx.reshape(-1),
                     E_pad.reshape(-1)).reshape(B * L * KPAD, QCOLS)

    nd = 2 * MAX_REL + 2
    Wcomb = W_pos @ W_edge[:NUM_PE]
    brow = (b_pos @ W_edge[:NUM_PE]).reshape(1, 128)
    W512 = jnp.zeros((FEXP, 128), jnp.float32)
    W512 = W512.at[:NPAIR * NUM_RBF].set(W_edge[NUM_PE:])
    W512 = W512.at[NPAIR * NUM_RBF:NPAIR * NUM_RBF + nd].set(Wcomb)
    D_mu = jnp.linspace(2.0, 22.0, NUM_RBF)
    mus = jnp.zeros((1, FEXP), jnp.float32)
    mus = mus.at[0, :NPAIR * NUM_RBF].set(jnp.tile(D_mu * 0.8, NPAIR))
    colv = jnp.full((1, FEXP), -1.0, jnp.float32)
    colv = colv.at[0, NPAIR * NUM_RBF:NPAIR * NUM_RBF + nd].set(
        jnp.arange(nd, dtype=jnp.float32))
    S2 = jnp.zeros((QCOLS, FEXP), jnp.float32)
    pcol = jnp.arange(NPAIR * NUM_RBF) // NUM_RBF
    S2 = S2.at[pcol, jnp.arange(NPAIR * NUM_RBF)].set(0.8)
    S2 = S2.at[NPAIR, NPAIR * NUM_RBF:NPAIR * NUM_RBF + nd].set(1.0)

    E = _run_edges(Q.reshape(B, L, KPAD, QCOLS), S2,
                   W512.astype(jnp.bfloat16), mus, colv, brow,
                   ln_scale.reshape(1, 128), ln_offset.reshape(1, 128))
    return (E, E_idx)


# speedup vs baseline: 1.1067x; 1.0084x over previous
import functools

import jax
import jax.numpy as jnp
from jax import lax
from jax.experimental import pallas as pl
from jax.experimental.pallas import tpu as pltpu
from jax.experimental.pallas import tpu_sc as plsc

TOP_K = 30
KPAD = 32
NUM_PE, NUM_RBF = 16, 16
MAX_REL = 32
NPAIR = 25
QCOLS = 32
FEXP = 512
EDGE_TILE = 1024


def _topk_body(x_ref, cat_ref, e_ref, t_ref, d_s, e_s):
    x = x_ref[0]
    n = x[:, 0:3]
    ca = x[:, 3:6]
    cc = x[:, 6:9]
    oo = x[:, 9:12]
    bv = ca - n
    cv = cc - ca
    b0, b1, b2 = bv[:, 0:1], bv[:, 1:2], bv[:, 2:3]
    c0, c1, c2 = cv[:, 0:1], cv[:, 1:2], cv[:, 2:3]
    cr = jnp.concatenate([b1 * c2 - b2 * c1,
                          b2 * c0 - b0 * c2,
                          b0 * c1 - b1 * c0], axis=1)
    cb = -0.58273431 * cr + 0.56802827 * bv - 0.54067466 * cv + ca
    zero_col = jnp.zeros((x.shape[0], 1), jnp.float32)
    t_ref[0] = jnp.concatenate([n, ca, cc, oo, cb, zero_col], axis=1)

    cat = cat_ref[0]
    acc = None
    for c in range(3):
        diff = x[:, 3 + c:4 + c] - cat[c:c + 1, :]
        sq = diff * diff
        acc = sq if acc is None else acc + sq
    d_s[...] = acc

    L = x.shape[0]
    iota_j = lax.broadcasted_iota(jnp.int32, (L, L), 0)

    def body(k, carry):
        dc = d_s[...]
        m = jnp.min(dc, axis=0, keepdims=True)
        cand = jnp.where(dc == m, iota_j, L * 2)
        idx = jnp.min(cand, axis=0, keepdims=True)
        e_s[pl.ds(k, 1), :] = idx
        d_s[...] = jnp.where(iota_j == idx, jnp.float32(jnp.inf), dc)
        return carry

    lax.fori_loop(0, TOP_K, body, 0)
    zrow = jnp.zeros((1, L), jnp.int32)
    e_s[pl.ds(TOP_K, 1), :] = zrow
    e_s[pl.ds(TOP_K + 1, 1), :] = zrow
    e_ref[0] = jnp.transpose(e_s[...], (1, 0))


def _run_topk(Xr, Cat):
    B, L = Xr.shape[0], Xr.shape[1]
    return pl.pallas_call(
        _topk_body,
        grid=(B,),
        in_specs=[
            pl.BlockSpec((1, L, 12), lambda b: (b, 0, 0)),
            pl.BlockSpec((1, 3, L), lambda b: (b, 0, 0)),
        ],
        out_specs=[
            pl.BlockSpec((1, L, KPAD), lambda b: (b, 0, 0)),
            pl.BlockSpec((1, L, 16), lambda b: (b, 0, 0)),
        ],
        out_shape=[
            jax.ShapeDtypeStruct((B, L, KPAD), jnp.int32),
            jax.ShapeDtypeStruct((B, L, 16), jnp.float32),
        ],
        scratch_shapes=[pltpu.VMEM((L, L), jnp.float32),
                        pltpu.VMEM((KPAD, L), jnp.int32)],
    )(Xr, Cat)


def _edge_body(q_ref, s2_ref, w_ref, mus_ref, colv_ref, brow_ref, sc_ref,
               of_ref, o_ref):
    lrows = q_ref.shape[1]
    q = q_ref[0].reshape(lrows * KPAD, QCOLS)
    c32 = lax.broadcasted_iota(jnp.int32, q.shape, 1)
    qmix = jnp.where(c32 < NPAIR, jnp.sqrt(q + 1e-6), q)
    dexp = lax.dot_general(qmix, s2_ref[...], (((1,), (0,)), ((), ())),
                           preferred_element_type=jnp.float32)
    t = dexp - mus_ref[...]
    rbf = jnp.exp(-(t * t))
    oneh = (dexp == colv_ref[...]).astype(jnp.float32)
    col = lax.broadcasted_iota(jnp.int32, dexp.shape, 1)
    a = jnp.where(col < NPAIR * NUM_RBF, rbf, oneh)
    acc = lax.dot_general(a, w_ref[...], (((1,), (0,)), ((), ())),
                          preferred_element_type=jnp.float32) + brow_ref[...]
    m = jnp.mean(acc, axis=1, keepdims=True)
    cen = acc - m
    v = jnp.mean(cen * cen, axis=1, keepdims=True)
    out = cen * lax.rsqrt(v + 1e-5) * sc_ref[...] + of_ref[...]
    o_ref[0] = out.reshape(lrows, KPAD, 128)[:, :TOP_K, :]


def _run_edges(Q4, S2, W512, mus, colv, brow, scale_row, off_row):
    B, L = Q4.shape[0], Q4.shape[1]
    lrows = EDGE_TILE // KPAD
    ntile = L // lrows
    return pl.pallas_call(
        _edge_body,
        grid=(B, ntile),
        in_specs=[
            pl.BlockSpec((1, lrows, KPAD, QCOLS), lambda b, e: (b, e, 0, 0)),
            pl.BlockSpec((QCOLS, FEXP), lambda b, e: (0, 0)),
            pl.BlockSpec((FEXP, 128), lambda b, e: (0, 0)),
            pl.BlockSpec((1, FEXP), lambda b, e: (0, 0)),
            pl.BlockSpec((1, FEXP), lambda b, e: (0, 0)),
            pl.BlockSpec((1, 128), lambda b, e: (0, 0)),
            pl.BlockSpec((1, 128), lambda b, e: (0, 0)),
            pl.BlockSpec((1, 128), lambda b, e: (0, 0)),
        ],
        out_specs=pl.BlockSpec((1, lrows, TOP_K, 128), lambda b, e: (b, e, 0, 0)),
        out_shape=jax.ShapeDtypeStruct((B, L, TOP_K, 128), jnp.float32),
    )(Q4, S2, W512, mus, colv, brow, scale_row, off_row)


_PAIRS_PY = [(1, 1), (0, 0), (2, 2), (3, 3), (4, 4), (1, 0), (1, 2), (1, 3),
             (1, 4), (0, 2), (0, 3), (0, 4), (4, 2), (4, 3), (3, 2), (0, 1),
             (2, 1), (3, 1), (4, 1), (2, 0), (3, 0), (4, 0), (2, 4), (3, 4),
             (2, 3)]

_NW = 32
_EDGES_PER_W = (4 * 512 * KPAD) // _NW


def _gather_q_sc(T_flat, chain_flat, e_flat):
    nrow = T_flat.shape[0] // 16
    nedge = e_flat.shape[0]
    mesh = plsc.VectorSubcoreMesh(core_axis_name="c", subcore_axis_name="s")

    @functools.partial(
        pl.kernel, mesh=mesh,
        compiler_params=pltpu.CompilerParams(needs_layout_passes=False),
        out_type=jax.ShapeDtypeStruct((nedge * QCOLS,), jnp.float32),
        scratch_types=[
            pltpu.VMEM((nrow * 16,), jnp.float32),
            pltpu.VMEM((nrow,), jnp.int32),
            pltpu.VMEM((_EDGES_PER_W,), jnp.int32),
            pltpu.VMEM((_EDGES_PER_W * QCOLS,), jnp.float32),
        ],
    )
    def k(t_hbm, ch_hbm, e_hbm, q_hbm, tv, chv, ev, qv):
        wid = lax.axis_index("s") * 2 + lax.axis_index("c")
        base = wid * _EDGES_PER_W
        pltpu.sync_copy(t_hbm, tv)
        pltpu.sync_copy(ch_hbm, chv)
        pltpu.sync_copy(e_hbm.at[pl.ds(base, _EDGES_PER_W)], ev)
        b512 = (base >> 14) << 9
        lane = jnp.arange(16, dtype=jnp.int32)
        zz = jnp.zeros((16,), jnp.float32)

        def body(g, carry):
            eg = g * 16 + lane
            j = plsc.load_gather(ev, [eg])
            gcen = (base + eg) >> 5
            gj = j + b512
            cc = plsc.load_gather(chv, [gcen])
            cn = plsc.load_gather(chv, [gj])
            off = gcen - gj
            dcl = jnp.clip(off + MAX_REL, 0, 2 * MAX_REL)
            dd = jnp.where(cc == cn, dcl, 2 * MAX_REL + 1).astype(jnp.float32)
            tc16 = gcen * 16
            tj16 = gj * 16
            ct = [plsc.load_gather(tv, [tc16 + c]) for c in range(15)]
            nb = [plsc.load_gather(tv, [tj16 + c]) for c in range(15)]
            qbase = eg * QCOLS
            for p, (ap, bp) in enumerate(_PAIRS_PY):
                acc = None
                for c in range(3):
                    dif = ct[3 * ap + c] - nb[3 * bp + c]
                    sq = dif * dif
                    acc = sq if acc is None else acc + sq
                plsc.store_scatter(qv, [qbase + p], acc)
            plsc.store_scatter(qv, [qbase + NPAIR], dd)
            for c in range(NPAIR + 1, QCOLS):
                plsc.store_scatter(qv, [qbase + c], zz)
            return carry

        lax.fori_loop(0, _EDGES_PER_W // 16, body, 0)
        pltpu.sync_copy(qv, q_hbm.at[pl.ds(base * QCOLS, _EDGES_PER_W * QCOLS)])

    return k(T_flat, chain_flat, e_flat)


def kernel(X, mask, residue_idx, chain_idx, W_pos, b_pos, W_edge, ln_scale, ln_offset):
    B, L = X.shape[0], X.shape[1]
    K = TOP_K
    Xr = X.reshape(B, L, 12)
    Cat = X[:, :, 1, :].transpose(0, 2, 1)
    E_pad, T = _run_topk(Xr, Cat)
    E_idx = E_pad[:, :, :K]

    Q = _gather_q_sc(T.reshape(-1), chain_idx.reshape(-1),
                     E_pad.reshape(-1)).reshape(B * L * KPAD, QCOLS)

    nd = 2 * MAX_REL + 2
    Wcomb = W_pos @ W_edge[:NUM_PE]
    brow = (b_pos @ W_edge[:NUM_PE]).reshape(1, 128)
    W512 = jnp.zeros((FEXP, 128), jnp.float32)
    W512 = W512.at[:NPAIR * NUM_RBF].set(W_edge[NUM_PE:])
    W512 = W512.at[NPAIR * NUM_RBF:NPAIR * NUM_RBF + nd].set(Wcomb)
    D_mu = jnp.linspace(2.0, 22.0, NUM_RBF)
    mus = jnp.zeros((1, FEXP), jnp.float32)
    mus = mus.at[0, :NPAIR * NUM_RBF].set(jnp.tile(D_mu * 0.8, NPAIR))
    colv = jnp.full((1, FEXP), -1.0, jnp.float32)
    colv = colv.at[0, NPAIR * NUM_RBF:NPAIR * NUM_RBF + nd].set(
        jnp.arange(nd, dtype=jnp.float32))
    S2 = jnp.zeros((QCOLS, FEXP), jnp.float32)
    pcol = jnp.arange(NPAIR * NUM_RBF) // NUM_RBF
    S2 = S2.at[pcol, jnp.arange(NPAIR * NUM_RBF)].set(0.8)
    S2 = S2.at[NPAIR, NPAIR * NUM_RBF:NPAIR * NUM_RBF + nd].set(1.0)

    E = _run_edges(Q.reshape(B, L, KPAD, QCOLS), S2, W512, mus, colv, brow,
                   ln_scale.reshape(1, 128), ln_offset.reshape(1, 128))
    return (E, E_idx)
